# baseline (speedup 1.0000x reference)
import functools

import jax
import jax.numpy as jnp
from jax import lax
from jax.experimental import pallas as pl
from jax.experimental.pallas import tpu as pltpu
from jax.experimental.pallas import tpu_sc as plsc

D = 128
B = 4096
L = 20
NSEQ = 6
R = NSEQ * B
NC = 2
NS = 16
NW = NC * NS
RW = R // NW
CHUNK = 120
MAXCH = RW * L // CHUNK
RING = 4
FLUSH = 64
OCAP = 128


def _pool_body(table_hbm, ids_hbm, lens_hbm, out_hbm,
               ids_v, lens_v, nrows_v, rows0, rows1, rows2, rows3, out_v,
               g0, g1, g2, g3, osem):
    wid = lax.axis_index("s") * NC + lax.axis_index("c")
    base_row = wid * RW
    rbufs = (rows0, rows1, rows2, rows3)
    gsems = (g0, g1, g2, g3)
    iota = lax.iota(jnp.int32, 16)
    zero16 = jnp.zeros((16,), jnp.float32)

    pltpu.sync_copy(ids_hbm.at[pl.ds(base_row * L, RW * L)],
                    ids_v.at[pl.ds(0, RW * L)])
    pltpu.sync_copy(lens_hbm.at[pl.ds(base_row, RW)],
                    lens_v.at[pl.ds(0, RW)])
    for blk in range((MAXCH + 16) // 16):
        nrows_v[pl.ds(blk * 16, 16)] = iota * 0

    def close_chunk(chunk, slot, rows):
        t = nrows_v[pl.ds(chunk, 16)]
        nrows_v[pl.ds(chunk, 16)] = jnp.where(iota == 0, rows, t)
        for blk in range(CHUNK // 16 + 1):
            pos = chunk * CHUNK + blk * 16
            u = ids_v[pl.ds(pos, 16)]
            ids_v[pl.ds(pos, 16)] = jnp.where(blk * 16 + iota >= slot, 0, u)

    def pack_row(r, carry):
        chunk, slot, rows = carry
        length = lens_v[pl.ds(r, 16)][0]
        v0 = ids_v[pl.ds(r * L, 16)]
        v1 = ids_v[pl.ds(r * L + 16, 16)]
        close = slot + length > CHUNK

        @pl.when(close)
        def _():
            close_chunk(chunk, slot, rows)

        chunk = chunk + close.astype(jnp.int32)
        slot = jnp.where(close, 0, slot)
        rows = jnp.where(close, 0, rows)
        pos = chunk * CHUNK + slot
        l0 = jnp.minimum(length, 16)
        plsc.store_scatter(ids_v, [iota + pos], v0, mask=iota < length)
        plsc.store_scatter(ids_v, [iota + (pos + l0)], v1,
                           mask=iota < length - 16)
        return chunk, slot + length, rows + 1

    chunk, slot, rows = lax.fori_loop(0, RW, pack_row,
                                      (jnp.int32(0), jnp.int32(0),
                                       jnp.int32(0)))
    close_chunk(chunk, slot, rows)
    nch = chunk + 1

    def start_gather(c, rows_buf, sem):
        cc = jnp.minimum(c, MAXCH - 1)
        idx = ids_v.at[pl.ds(cc * CHUNK, CHUNK)]
        pltpu.async_copy(table_hbm.at[idx], rows_buf, sem)

    def wait_gather(rows_buf, sem):
        pltpu.make_async_copy(table_hbm.at[pl.ds(0, CHUNK)], rows_buf,
                              sem).wait()

    def start_flush(rp):
        pltpu.async_copy(
            out_v.at[pl.ds((rp % OCAP) * D, FLUSH * D)],
            out_hbm.at[pl.ds((base_row + rp) * D, FLUSH * D)], osem)

    def wait_flush():
        pltpu.make_async_copy(out_v.at[pl.ds(0, FLUSH * D)],
                              out_hbm.at[pl.ds(0, FLUSH * D)], osem).wait()

    for b in range(RING):
        start_gather(jnp.int32(b), rbufs[b], gsems[b])

    def consume_chunk(c, rows_buf, carry):
        r, wp, rp = carry
        nrows_c = nrows_v[pl.ds(c, 16)][0]

        def row_body(_, rc):
            r, pos, wp, rp = rc
            length = lens_v[pl.ds(r, 16)][0]

            def jbody(j, acc):
                return tuple(
                    acc[d] + rows_buf[pos + j, pl.ds(d * 16, 16)]
                    for d in range(D // 16))

            acc = lax.fori_loop(0, length, jbody, (zero16,) * (D // 16))
            obase = (wp % OCAP) * D
            for d in range(D // 16):
                out_v[pl.ds(obase + d * 16, 16)] = acc[d] * (1.0 / L)
            wp = wp + 1
            flush = wp - rp >= FLUSH

            @pl.when(flush)
            def _():
                @pl.when(rp > 0)
                def _():
                    wait_flush()

                start_flush(rp)

            rp = rp + jnp.where(flush, FLUSH, 0)
            return r + 1, pos + length, wp, rp

        r, _, wp, rp = lax.fori_loop(0, nrows_c, row_body,
                                     (r, jnp.int32(0), wp, rp))
        return r, wp, rp

    def outer(co, carry):
        for b in range(RING):
            c = co * RING + b
            wait_gather(rbufs[b], gsems[b])
            carry = consume_chunk(c, rbufs[b], carry)
            start_gather(c + RING, rbufs[b], gsems[b])
        return carry

    trips = (nch + RING - 1) // RING
    lax.fori_loop(0, trips, outer,
                  (jnp.int32(0), jnp.int32(0), jnp.int32(0)))
    wait_flush()
    for b in range(RING):
        wait_gather(rbufs[b], gsems[b])


@functools.partial(
    pl.kernel,
    mesh=plsc.VectorSubcoreMesh(core_axis_name="c", subcore_axis_name="s"),
    compiler_params=pltpu.CompilerParams(needs_layout_passes=False),
    out_type=jax.ShapeDtypeStruct((R * D,), jnp.float32),
    scratch_types=[
        pltpu.VMEM((RW * L + 32,), jnp.int32),
        pltpu.VMEM((RW + 16,), jnp.int32),
        pltpu.VMEM((MAXCH + 16,), jnp.int32),
        pltpu.VMEM((CHUNK, D), jnp.float32),
        pltpu.VMEM((CHUNK, D), jnp.float32),
        pltpu.VMEM((CHUNK, D), jnp.float32),
        pltpu.VMEM((CHUNK, D), jnp.float32),
        pltpu.VMEM((OCAP * D,), jnp.float32),
        pltpu.SemaphoreType.DMA,
        pltpu.SemaphoreType.DMA,
        pltpu.SemaphoreType.DMA,
        pltpu.SemaphoreType.DMA,
        pltpu.SemaphoreType.DMA,
    ],
)
def _pool(table_hbm, ids_hbm, lens_hbm, out_hbm, *rest):
    _pool_body(table_hbm, ids_hbm, lens_hbm, out_hbm, *rest)


def _fc_body(x_ref, w_ref, b_ref, o_ref):
    w = w_ref[...]
    acc = b_ref[0][None, :].astype(jnp.float32)
    for p in range(3):
        acc = acc + jax.lax.dot_general(
            x_ref[0, p], w[p * D:(p + 1) * D, :],
            (((1,), (0,)), ((), ())),
            preferred_element_type=jnp.float32,
            precision=jax.lax.Precision.HIGHEST,
        )
    o_ref[0] = acc


_RB = 512

_fc = pl.pallas_call(
    _fc_body,
    grid=(2, B // _RB),
    in_specs=[
        pl.BlockSpec((1, 3, _RB, D), lambda e, r: (e, 0, r, 0)),
        pl.BlockSpec((3 * D, D), lambda e, r: (0, 0)),
        pl.BlockSpec((1, D), lambda e, r: (0, 0)),
    ],
    out_specs=pl.BlockSpec((1, _RB, D), lambda e, r: (e, r, 0)),
    out_shape=jax.ShapeDtypeStruct((2, B, D), jnp.float32),
)


def kernel(table, fc_w, fc_b,
           evtq_s_ids, evtq_s_lengths, evtq_p_ids, evtq_p_lengths,
           evtq_o_ids, evtq_o_lengths,
           evtk_s_ids, evtk_s_lengths, evtk_p_ids, evtk_p_lengths,
           evtk_o_ids, evtk_o_lengths):
    ids_all = jnp.stack([evtq_s_ids, evtq_p_ids, evtq_o_ids,
                         evtk_s_ids, evtk_p_ids, evtk_o_ids])
    lens_all = jnp.stack([evtq_s_lengths, evtq_p_lengths, evtq_o_lengths,
                          evtk_s_lengths, evtk_p_lengths, evtk_o_lengths])
    pooled = _pool(table, ids_all.reshape(-1).astype(jnp.int32),
                   lens_all.reshape(-1).astype(jnp.int32))
    out2 = _fc(pooled.reshape(2, 3, B, D), fc_w, fc_b.reshape(1, D))
    return out2[0], out2[1]


# speedup vs baseline: 3.6259x; 3.6259x over previous
import functools

import jax
import jax.numpy as jnp
from jax import lax
from jax.experimental import pallas as pl
from jax.experimental.pallas import tpu as pltpu
from jax.experimental.pallas import tpu_sc as plsc

D = 128
B = 4096
L = 20
NSEQ = 6
R = NSEQ * B
NC = 2
NS = 16
NW = NC * NS
RW = R // NW
G = 4
NIDX = G * L
STEPS = RW // G
NDBLK = D // 16


NBUF = 8
OSTEP = NBUF * G


def _pool_body(table_hbm, ids_hbm, coef_hbm, out_hbm,
               ids_v, coef_v, rows0, rows1, rows2, rows3, rows4, rows5,
               rows6, rows7, out_v, g0, g1, g2, g3, g4, g5, g6, g7, osem):
    wid = lax.axis_index("s") * NC + lax.axis_index("c")
    base_row = wid * RW
    rbufs = (rows0, rows1, rows2, rows3, rows4, rows5, rows6, rows7)
    gsems = (g0, g1, g2, g3, g4, g5, g6, g7)

    pltpu.sync_copy(ids_hbm.at[pl.ds(base_row * L, RW * L)], ids_v)
    pltpu.sync_copy(coef_hbm.at[pl.ds(base_row, RW)], coef_v.at[pl.ds(0, RW)])

    def start_gather(s, rows_buf, sem):
        idx = ids_v.at[pl.ds(s * NIDX, NIDX)]
        pltpu.async_copy(table_hbm.at[idx], rows_buf, sem)

    def wait_gather(rows_buf, sem):
        pltpu.make_async_copy(table_hbm.at[pl.ds(0, NIDX)], rows_buf, sem).wait()

    def start_out(so):
        pltpu.async_copy(
            out_v, out_hbm.at[pl.ds((base_row + so * OSTEP) * D, OSTEP * D)],
            osem)

    def wait_out():
        pltpu.make_async_copy(
            out_v, out_hbm.at[pl.ds(0, OSTEP * D)], osem).wait()

    def compute(s, b, rows_buf):
        cvec = coef_v[pl.ds(s * G, 16)]
        for i in range(G):
            c = cvec[i]

            def dbody(dblk, _):
                sl = pl.ds(dblk * 16, 16)
                accs = [None] * 4
                e0 = None
                for j in range(L):
                    v = rows_buf[i * L + j, sl]
                    if j == 0:
                        e0 = v
                    k = j % 4
                    accs[k] = v if accs[k] is None else accs[k] + v
                acc = (accs[0] + accs[1]) + (accs[2] + accs[3])
                out_v[pl.ds((b * G + i) * D + dblk * 16, 16)] = (
                    acc * (1.0 / L) + c * e0)
                return _

            lax.fori_loop(0, NDBLK, dbody, 0)

    for b in range(NBUF):
        start_gather(b, rbufs[b], gsems[b])

    def body(so, carry):
        @pl.when(so >= 1)
        def _():
            wait_out()

        for b in range(NBUF):
            s = so * NBUF + b
            wait_gather(rbufs[b], gsems[b])
            compute(s, b, rbufs[b])

            @pl.when(s + NBUF < STEPS)
            def _():
                start_gather(s + NBUF, rbufs[b], gsems[b])

        start_out(so)
        return carry

    lax.fori_loop(0, STEPS // NBUF, body, 0)
    wait_out()


@functools.partial(
    pl.kernel,
    mesh=plsc.VectorSubcoreMesh(core_axis_name="c", subcore_axis_name="s"),
    out_type=jax.ShapeDtypeStruct((R * D,), jnp.float32),
    scratch_types=[
        pltpu.VMEM((RW * L,), jnp.int32),
        pltpu.VMEM((RW + 16,), jnp.float32),
        pltpu.VMEM((NIDX, D), jnp.float32),
        pltpu.VMEM((NIDX, D), jnp.float32),
        pltpu.VMEM((NIDX, D), jnp.float32),
        pltpu.VMEM((NIDX, D), jnp.float32),
        pltpu.VMEM((NIDX, D), jnp.float32),
        pltpu.VMEM((NIDX, D), jnp.float32),
        pltpu.VMEM((NIDX, D), jnp.float32),
        pltpu.VMEM((NIDX, D), jnp.float32),
        pltpu.VMEM((OSTEP * D,), jnp.float32),
        pltpu.SemaphoreType.DMA,
        pltpu.SemaphoreType.DMA,
        pltpu.SemaphoreType.DMA,
        pltpu.SemaphoreType.DMA,
        pltpu.SemaphoreType.DMA,
        pltpu.SemaphoreType.DMA,
        pltpu.SemaphoreType.DMA,
        pltpu.SemaphoreType.DMA,
        pltpu.SemaphoreType.DMA,
    ],
)
def _pool(table_hbm, ids_hbm, coef_hbm, out_hbm, *rest):
    _pool_body(table_hbm, ids_hbm, coef_hbm, out_hbm, *rest)


def _fc_body(x_ref, w_ref, b_ref, o_ref):
    w = w_ref[...]
    acc = b_ref[0][None, :].astype(jnp.float32)
    for p in range(3):
        acc = acc + jax.lax.dot_general(
            x_ref[0, p], w[p * D:(p + 1) * D, :],
            (((1,), (0,)), ((), ())),
            preferred_element_type=jnp.float32,
            precision=jax.lax.Precision.HIGHEST,
        )
    o_ref[0] = acc


_RB = 512

_fc = pl.pallas_call(
    _fc_body,
    grid=(2, B // _RB),
    in_specs=[
        pl.BlockSpec((1, 3, _RB, D), lambda e, r: (e, 0, r, 0)),
        pl.BlockSpec((3 * D, D), lambda e, r: (0, 0)),
        pl.BlockSpec((1, D), lambda e, r: (0, 0)),
    ],
    out_specs=pl.BlockSpec((1, _RB, D), lambda e, r: (e, r, 0)),
    out_shape=jax.ShapeDtypeStruct((2, B, D), jnp.float32),
)


def kernel(table, fc_w, fc_b,
           evtq_s_ids, evtq_s_lengths, evtq_p_ids, evtq_p_lengths,
           evtq_o_ids, evtq_o_lengths,
           evtk_s_ids, evtk_s_lengths, evtk_p_ids, evtk_p_lengths,
           evtk_o_ids, evtk_o_lengths):
    ids_all = jnp.stack([evtq_s_ids, evtq_p_ids, evtq_o_ids,
                         evtk_s_ids, evtk_p_ids, evtk_o_ids])
    lens_all = jnp.stack([evtq_s_lengths, evtq_p_lengths, evtq_o_lengths,
                          evtk_s_lengths, evtk_p_lengths, evtk_o_lengths])
    pos = jnp.arange(L, dtype=lens_all.dtype)
    idsm = jnp.where(pos[None, None, :] < lens_all[:, :, None],
                     ids_all, ids_all[:, :, :1]).astype(jnp.int32)
    coef = (lens_all.astype(jnp.float32) - L) * (1.0 / L)

    pooled = _pool(table, idsm.reshape(-1), coef.reshape(-1))
    out2 = _fc(pooled.reshape(2, 3, B, D), fc_w, fc_b.reshape(1, D))
    return out2[0], out2[1]
